# Initial kernel scaffold; baseline (speedup 1.0000x reference)
import jax
import jax.numpy as jnp
from jax.experimental import pallas as pl


def kernel(hidden, expert_ids, gate_up, down):
    raise NotImplementedError("write your pallas kernel here")


# speedup vs baseline: 1.7684x; 1.7684x over previous
import functools

import jax
import jax.numpy as jnp
from jax.experimental import pallas as pl
from jax.experimental.pallas import tpu as pltpu

E = 8
D = 2048
I = 2048
T = 4096

BLK = 256
IBLK = 512
NJ = I // IBLK
NBMAX = T // BLK + E
TPAD = NBMAX * BLK


def _route_metadata(ids):
    ids = ids.astype(jnp.int32)
    counts = jnp.sum(ids[None, :] == jnp.arange(E, dtype=jnp.int32)[:, None], axis=1)
    padded = ((counts + BLK - 1) // BLK) * BLK
    pad_end = jnp.cumsum(padded)
    pad_off = pad_end - padded
    order = jnp.argsort(ids, stable=True)
    sorted_ids = ids[order]
    start_sorted = jnp.cumsum(counts) - counts
    rank_sorted = jnp.arange(T, dtype=jnp.int32) - start_sorted[sorted_ids]
    pos_sorted = pad_off[sorted_ids] + rank_sorted
    pos = jnp.zeros((T,), jnp.int32).at[order].set(pos_sorted)
    src = jnp.zeros((TPAD,), jnp.int32).at[pos_sorted].set(order.astype(jnp.int32))
    blocks = jnp.arange(NBMAX, dtype=jnp.int32) * BLK
    block_expert = jnp.minimum(
        jnp.sum(blocks[:, None] >= pad_end[None, :], axis=1), E - 1
    ).astype(jnp.int32)
    return pos, src, block_expert


def _mm_body(be_ref, x_ref, wg_ref, wu_ref, wd_ref, o_ref):
    j = pl.program_id(1)
    x = x_ref[...]
    g = jnp.dot(x, wg_ref[0], preferred_element_type=jnp.float32)
    u = jnp.dot(x, wu_ref[0], preferred_element_type=jnp.float32)
    act = g * jax.nn.sigmoid(g) * u
    p = jnp.dot(act, wd_ref[0], preferred_element_type=jnp.float32)

    @pl.when(j == 0)
    def _():
        o_ref[...] = p

    @pl.when(j > 0)
    def _():
        o_ref[...] += p


def _grouped_mlp(xpad, gate_up, down, block_expert, interpret=False):
    grid_spec = pltpu.PrefetchScalarGridSpec(
        num_scalar_prefetch=1,
        grid=(NBMAX, NJ),
        in_specs=[
            pl.BlockSpec((BLK, D), lambda i, j, be: (i, 0)),
            pl.BlockSpec((1, D, IBLK), lambda i, j, be: (be[i], 0, j)),
            pl.BlockSpec((1, D, IBLK), lambda i, j, be: (be[i], 0, NJ + j)),
            pl.BlockSpec((1, IBLK, D), lambda i, j, be: (be[i], j, 0)),
        ],
        out_specs=pl.BlockSpec((BLK, D), lambda i, j, be: (i, 0)),
    )
    return pl.pallas_call(
        _mm_body,
        grid_spec=grid_spec,
        out_shape=jax.ShapeDtypeStruct((TPAD, D), jnp.float32),
        compiler_params=pltpu.CompilerParams(
            dimension_semantics=("arbitrary", "arbitrary"),
        ),
        interpret=interpret,
    )(block_expert, xpad, gate_up, gate_up, down)


@jax.jit
def kernel(hidden, expert_ids, gate_up, down):
    pos, src, block_expert = _route_metadata(expert_ids)
    xpad = jnp.take(hidden, src, axis=0)
    opad = _grouped_mlp(xpad, gate_up, down, block_expert)
    return jnp.take(opad, pos, axis=0)


# speedup vs baseline: 2.0160x; 1.1400x over previous
import functools

import jax
import jax.numpy as jnp
from jax import lax
from jax.experimental import pallas as pl
from jax.experimental.pallas import tpu as pltpu
from jax.experimental.pallas import tpu_sc as plsc

E = 8
D = 2048
I = 2048
T = 4096

BLK = 256
BLK_SHIFT = 8
IBLK = 512
NJ = I // IBLK
NBMAX = T // BLK + E
TPAD = NBMAX * BLK

NC = 2
NS = 16
NW = NC * NS
L = 16
GCH = 32

_SC_MESH = functools.partial(
    plsc.VectorSubcoreMesh, core_axis_name="c", subcore_axis_name="s"
)


def _route_body(ids_hbm, pos_hbm, src_hbm, be_hbm,
                ids_v, rank_v, pos_v, src_v, run_v, off_v, be_v):
    wid = lax.axis_index("s") * NC + lax.axis_index("c")

    @pl.when(wid == 0)
    def _():
        pltpu.sync_copy(ids_hbm, ids_v)
        lane = lax.iota(jnp.int32, L)
        zeros = jnp.zeros((L,), jnp.int32)
        run_v[...] = zeros

        def p1(c, _):
            v = ids_v[pl.ds(c * L, L)]
            base = plsc.load_gather(run_v, [v])
            dup, last = plsc.scan_count(v)
            rank_v[pl.ds(c * L, L)] = base + dup - 1
            plsc.store_scatter(run_v, [v], base + dup, mask=last)
            return 0

        lax.fori_loop(0, T // L, p1, 0)

        counts = run_v[...]
        padded = ((counts + (BLK - 1)) >> BLK_SHIFT) << BLK_SHIFT
        end = plsc.cumsum(padded)
        off_v[...] = end - padded

        def z(c, _):
            src_v[pl.ds(c * L, L)] = zeros
            return 0

        lax.fori_loop(0, TPAD // L, z, 0)

        def p2(c, _):
            v = ids_v[pl.ds(c * L, L)]
            offs = plsc.load_gather(off_v, [v])
            p = offs + rank_v[pl.ds(c * L, L)]
            pos_v[pl.ds(c * L, L)] = p
            plsc.store_scatter(src_v, [p], lane + c * L)
            return 0

        lax.fori_loop(0, T // L, p2, 0)

        for cb in range(2):
            blkpos = (lane + cb * L) * BLK
            bev = zeros
            for e in range(E):
                end_e = jnp.sum(jnp.where(lane == e, end, 0))
                bev = bev + jnp.where(blkpos >= end_e, 1, 0)
            be_v[pl.ds(cb * L, L)] = jnp.minimum(bev, E - 1)

        pltpu.sync_copy(pos_v, pos_hbm)
        pltpu.sync_copy(src_v, src_hbm)
        pltpu.sync_copy(be_v, be_hbm)


def _route(ids, interpret=False):
    return pl.kernel(
        _route_body,
        out_type=(
            jax.ShapeDtypeStruct((T,), jnp.int32),
            jax.ShapeDtypeStruct((TPAD,), jnp.int32),
            jax.ShapeDtypeStruct((2 * L,), jnp.int32),
        ),
        mesh=_SC_MESH(),
        scratch_types=[
            pltpu.VMEM((T,), jnp.int32),
            pltpu.VMEM((T,), jnp.int32),
            pltpu.VMEM((T,), jnp.int32),
            pltpu.VMEM((TPAD,), jnp.int32),
            pltpu.VMEM((L,), jnp.int32),
            pltpu.VMEM((L,), jnp.int32),
            pltpu.VMEM((2 * L,), jnp.int32),
        ],
        compiler_params=pltpu.CompilerParams(needs_layout_passes=False),
        interpret=interpret,
    )(ids)


def _gather_rows(table, idx, n_rows, interpret=False):
    rpw = n_rows // NW
    nch = rpw // GCH

    def body(table_hbm, idx_hbm, out_hbm, idx_v, rows_v, sem):
        wid = lax.axis_index("s") * NC + lax.axis_index("c")
        base = wid * rpw

        def chunk(i, _):
            b = base + i * GCH
            pltpu.sync_copy(idx_hbm.at[pl.ds(b, GCH)], idx_v)
            pltpu.async_copy(table_hbm.at[idx_v], rows_v, sem).wait()
            pltpu.sync_copy(rows_v, out_hbm.at[pl.ds(b, GCH)])
            return 0

        lax.fori_loop(0, nch, chunk, 0)

    return pl.kernel(
        body,
        out_type=jax.ShapeDtypeStruct((n_rows, D), jnp.float32),
        mesh=_SC_MESH(),
        scratch_types=[
            pltpu.VMEM((GCH,), jnp.int32),
            pltpu.VMEM((GCH, D), jnp.float32),
            pltpu.SemaphoreType.DMA,
        ],
        interpret=interpret,
    )(table, idx)


def _mm_body(be_ref, x_ref, wg_ref, wu_ref, wd_ref, o_ref):
    j = pl.program_id(1)
    x = x_ref[...]
    g = jnp.dot(x, wg_ref[0], preferred_element_type=jnp.float32)
    u = jnp.dot(x, wu_ref[0], preferred_element_type=jnp.float32)
    act = g * jax.nn.sigmoid(g) * u
    p = jnp.dot(act, wd_ref[0], preferred_element_type=jnp.float32)

    @pl.when(j == 0)
    def _():
        o_ref[...] = p

    @pl.when(j > 0)
    def _():
        o_ref[...] += p


def _grouped_mlp(xpad, gate_up, down, block_expert, interpret=False):
    grid_spec = pltpu.PrefetchScalarGridSpec(
        num_scalar_prefetch=1,
        grid=(NBMAX, NJ),
        in_specs=[
            pl.BlockSpec((BLK, D), lambda i, j, be: (i, 0)),
            pl.BlockSpec((1, D, IBLK), lambda i, j, be: (be[i], 0, j)),
            pl.BlockSpec((1, D, IBLK), lambda i, j, be: (be[i], 0, NJ + j)),
            pl.BlockSpec((1, IBLK, D), lambda i, j, be: (be[i], j, 0)),
        ],
        out_specs=pl.BlockSpec((BLK, D), lambda i, j, be: (i, 0)),
    )
    return pl.pallas_call(
        _mm_body,
        grid_spec=grid_spec,
        out_shape=jax.ShapeDtypeStruct((TPAD, D), jnp.float32),
        compiler_params=pltpu.CompilerParams(
            dimension_semantics=("arbitrary", "arbitrary"),
        ),
        interpret=interpret,
    )(block_expert, xpad, gate_up, gate_up, down)


@jax.jit
def kernel(hidden, expert_ids, gate_up, down):
    ids = expert_ids.astype(jnp.int32)
    pos, src, be = _route(ids)
    xpad = _gather_rows(hidden, src, TPAD)
    opad = _grouped_mlp(xpad, gate_up, down, be[:NBMAX])
    return _gather_rows(opad, pos, T)


# speedup vs baseline: 2.8868x; 1.4320x over previous
import functools

import jax
import jax.numpy as jnp
from jax import lax
from jax.experimental import pallas as pl
from jax.experimental.pallas import tpu as pltpu
from jax.experimental.pallas import tpu_sc as plsc

E = 8
D = 2048
I = 2048
T = 4096

BLK = 512
BLK_SHIFT = 9
IBLK = 512
NJ = I // IBLK
NBMAX = T // BLK + E
TPAD = NBMAX * BLK

NC = 2
NS = 16
NW = NC * NS
L = 16
GCH = 32

_SC_MESH = functools.partial(
    plsc.VectorSubcoreMesh, core_axis_name="c", subcore_axis_name="s"
)


def _route_body(ids_hbm, pos_hbm, src_hbm, be_hbm,
                ids_v, rank_v, pos_v, src_v, run_v, off_v, be_v):
    wid = lax.axis_index("s") * NC + lax.axis_index("c")

    @pl.when(wid == 0)
    def _():
        pltpu.sync_copy(ids_hbm, ids_v)
        lane = lax.iota(jnp.int32, L)
        zeros = jnp.zeros((L,), jnp.int32)
        run_v[...] = zeros

        def p1(c, _):
            v = ids_v[pl.ds(c * L, L)]
            base = plsc.load_gather(run_v, [v])
            dup, last = plsc.scan_count(v)
            rank_v[pl.ds(c * L, L)] = base + dup - 1
            plsc.store_scatter(run_v, [v], base + dup, mask=last)
            return 0

        lax.fori_loop(0, T // L, p1, 0)

        counts = run_v[...]
        padded = ((counts + (BLK - 1)) >> BLK_SHIFT) << BLK_SHIFT
        end = plsc.cumsum(padded)
        off_v[...] = end - padded

        def z(c, _):
            src_v[pl.ds(c * L, L)] = (lane + c * L) & (T - 1)
            return 0

        lax.fori_loop(0, TPAD // L, z, 0)

        def p2(c, _):
            v = ids_v[pl.ds(c * L, L)]
            offs = plsc.load_gather(off_v, [v])
            p = offs + rank_v[pl.ds(c * L, L)]
            pos_v[pl.ds(c * L, L)] = p
            plsc.store_scatter(src_v, [p], lane + c * L)
            return 0

        lax.fori_loop(0, T // L, p2, 0)

        nb = jnp.sum(end * jnp.where(lane == E - 1, 1, 0)) >> BLK_SHIFT
        for cb in range(2):
            blkpos = (lane + cb * L) * BLK
            bev = zeros
            for e in range(E):
                end_e = jnp.sum(jnp.where(lane == e, end, 0))
                bev = bev + jnp.where(blkpos >= end_e, 1, 0)
            bev = jnp.minimum(bev, E - 1)
            if cb == 1:
                bev = jnp.where(lane == NBMAX - L, nb, bev)
            be_v[pl.ds(cb * L, L)] = bev

        pltpu.sync_copy(pos_v, pos_hbm)
        pltpu.sync_copy(src_v, src_hbm)
        pltpu.sync_copy(be_v, be_hbm)


def _route(ids):
    return pl.kernel(
        _route_body,
        out_type=(
            jax.ShapeDtypeStruct((T,), jnp.int32),
            jax.ShapeDtypeStruct((TPAD,), jnp.int32),
            jax.ShapeDtypeStruct((2 * L,), jnp.int32),
        ),
        mesh=_SC_MESH(),
        scratch_types=[
            pltpu.VMEM((T,), jnp.int32),
            pltpu.VMEM((T,), jnp.int32),
            pltpu.VMEM((T,), jnp.int32),
            pltpu.VMEM((TPAD,), jnp.int32),
            pltpu.VMEM((L,), jnp.int32),
            pltpu.VMEM((L,), jnp.int32),
            pltpu.VMEM((2 * L,), jnp.int32),
        ],
        compiler_params=pltpu.CompilerParams(needs_layout_passes=False),
    )(ids)


def _gather_rows(table, idx, n_rows):
    rpw = n_rows // NW
    nch = rpw // GCH

    def body(table_hbm, idx_hbm, out_hbm, idx_v, rows_v, sem):
        wid = lax.axis_index("s") * NC + lax.axis_index("c")
        base = wid * rpw

        def chunk(i, _):
            b = base + i * GCH
            pltpu.sync_copy(idx_hbm.at[pl.ds(b, GCH)], idx_v)
            pltpu.async_copy(table_hbm.at[idx_v], rows_v, sem).wait()
            pltpu.sync_copy(rows_v, out_hbm.at[pl.ds(b, GCH)])
            return 0

        lax.fori_loop(0, nch, chunk, 0)

    return pl.kernel(
        body,
        out_type=jax.ShapeDtypeStruct((n_rows, D), jnp.float32),
        mesh=_SC_MESH(),
        scratch_types=[
            pltpu.VMEM((GCH,), jnp.int32),
            pltpu.VMEM((GCH, D), jnp.float32),
            pltpu.SemaphoreType.DMA,
        ],
    )(table, idx)


def _mm_body(be_ref, nb_ref, x_ref, wg_ref, wu_ref, wd_ref, o_ref):
    i = pl.program_id(0)
    j = pl.program_id(1)

    @pl.when(i < nb_ref[0])
    def _():
        x = x_ref[...]
        g = jnp.dot(x, wg_ref[0], preferred_element_type=jnp.float32)
        u = jnp.dot(x, wu_ref[0], preferred_element_type=jnp.float32)
        act = g * jax.nn.sigmoid(g) * u
        p = jnp.dot(act, wd_ref[0], preferred_element_type=jnp.float32)

        @pl.when(j == 0)
        def _():
            o_ref[...] = p

        @pl.when(j > 0)
        def _():
            o_ref[...] += p


def _grouped_mlp(xpad, gate_up, down, block_expert, nb):
    grid_spec = pltpu.PrefetchScalarGridSpec(
        num_scalar_prefetch=2,
        grid=(NBMAX, NJ),
        in_specs=[
            pl.BlockSpec((BLK, D), lambda i, j, be, nb: (i, 0)),
            pl.BlockSpec((1, D, IBLK), lambda i, j, be, nb: (be[i], 0, j)),
            pl.BlockSpec((1, D, IBLK), lambda i, j, be, nb: (be[i], 0, NJ + j)),
            pl.BlockSpec((1, IBLK, D), lambda i, j, be, nb: (be[i], j, 0)),
        ],
        out_specs=pl.BlockSpec((BLK, D), lambda i, j, be, nb: (i, 0)),
    )
    return pl.pallas_call(
        _mm_body,
        grid_spec=grid_spec,
        out_shape=jax.ShapeDtypeStruct((TPAD, D), jnp.float32),
        compiler_params=pltpu.CompilerParams(
            dimension_semantics=("arbitrary", "arbitrary"),
        ),
    )(block_expert, nb, xpad, gate_up, gate_up, down)


@jax.jit
def kernel(hidden, expert_ids, gate_up, down):
    ids = expert_ids.astype(jnp.int32)
    pos, src, meta = _route(ids)
    xpad = _gather_rows(hidden, src, TPAD)
    opad = _grouped_mlp(xpad, gate_up, down, meta[:NBMAX], meta[NBMAX:NBMAX + 1])
    return _gather_rows(opad, pos, T)


# speedup vs baseline: 2.9240x; 1.0129x over previous
import functools

import jax
import jax.numpy as jnp
from jax import lax
from jax.experimental import pallas as pl
from jax.experimental.pallas import tpu as pltpu
from jax.experimental.pallas import tpu_sc as plsc

E = 8
D = 2048
I = 2048
T = 4096

BLK = 1024
BLK_SHIFT = 10
HB = 512
IBLK = 256
NJ = I // IBLK
NBMAX = T // BLK + E
TPAD = NBMAX * BLK

NC = 2
NS = 16
NW = NC * NS
L = 16
GCH = 32

_SC_MESH = functools.partial(
    plsc.VectorSubcoreMesh, core_axis_name="c", subcore_axis_name="s"
)


def _route_body(ids_hbm, pos_hbm, src_hbm, be_hbm,
                ids_v, rank_v, pos_v, src_v, run_v, off_v, be_v):
    wid = lax.axis_index("s") * NC + lax.axis_index("c")

    @pl.when(wid == 0)
    def _():
        pltpu.sync_copy(ids_hbm, ids_v)
        lane = lax.iota(jnp.int32, L)
        zeros = jnp.zeros((L,), jnp.int32)
        run_v[...] = zeros

        def p1(c, _):
            v = ids_v[pl.ds(c * L, L)]
            base = plsc.load_gather(run_v, [v])
            dup, last = plsc.scan_count(v)
            rank_v[pl.ds(c * L, L)] = base + dup - 1
            plsc.store_scatter(run_v, [v], base + dup, mask=last)
            return 0

        lax.fori_loop(0, T // L, p1, 0)

        counts = run_v[...]
        padded = ((counts + (BLK - 1)) >> BLK_SHIFT) << BLK_SHIFT
        end = plsc.cumsum(padded)
        off_v[...] = end - padded

        def z(c, _):
            src_v[pl.ds(c * L, L)] = (lane + c * L) & (T - 1)
            return 0

        lax.fori_loop(0, TPAD // L, z, 0)

        def p2(c, _):
            v = ids_v[pl.ds(c * L, L)]
            offs = plsc.load_gather(off_v, [v])
            p = offs + rank_v[pl.ds(c * L, L)]
            pos_v[pl.ds(c * L, L)] = p
            plsc.store_scatter(src_v, [p], lane + c * L)
            return 0

        lax.fori_loop(0, T // L, p2, 0)

        nb = jnp.sum(end * jnp.where(lane == E - 1, 1, 0)) >> BLK_SHIFT
        off_v[...] = (end - padded) + counts
        blkpos = lane * BLK
        bev = zeros
        for e in range(E):
            end_e = jnp.sum(jnp.where(lane == e, end, 0))
            bev = bev + jnp.where(blkpos >= end_e, 1, 0)
        bev = jnp.minimum(bev, E - 1)
        realend = plsc.load_gather(off_v, [bev])
        used = jnp.minimum(jnp.maximum(realend - blkpos, 0), BLK)
        be_v[pl.ds(0, L)] = jnp.where(lane == NBMAX, nb, bev)
        be_v[pl.ds(L, L)] = used
        pltpu.sync_copy(pos_v, pos_hbm)
        pltpu.sync_copy(src_v, src_hbm)
        pltpu.sync_copy(be_v, be_hbm)


def _route(ids):
    return pl.kernel(
        _route_body,
        out_type=(
            jax.ShapeDtypeStruct((T,), jnp.int32),
            jax.ShapeDtypeStruct((TPAD,), jnp.int32),
            jax.ShapeDtypeStruct((2 * L,), jnp.int32),
        ),
        mesh=_SC_MESH(),
        scratch_types=[
            pltpu.VMEM((T,), jnp.int32),
            pltpu.VMEM((T,), jnp.int32),
            pltpu.VMEM((T,), jnp.int32),
            pltpu.VMEM((TPAD,), jnp.int32),
            pltpu.VMEM((L,), jnp.int32),
            pltpu.VMEM((L,), jnp.int32),
            pltpu.VMEM((2 * L,), jnp.int32),
        ],
        compiler_params=pltpu.CompilerParams(needs_layout_passes=False),
    )(ids)


def _gather_rows(table, idx, n_rows, meta=None):
    rpw = n_rows // NW
    nch = rpw // GCH

    def body(table_hbm, idx_hbm, meta_hbm, out_hbm, idx_v, rows_v, m_v, sem):
        wid = lax.axis_index("s") * NC + lax.axis_index("c")
        base = wid * rpw
        lane = lax.iota(jnp.int32, L)
        pltpu.sync_copy(meta_hbm, m_v)
        nbrows = jnp.sum(jnp.where(lane == NBMAX, m_v[pl.ds(0, L)], 0)) * BLK

        def chunk(i, _):
            b = base + i * GCH

            @pl.when(b < nbrows)
            def _():
                pltpu.sync_copy(idx_hbm.at[pl.ds(b, GCH)], idx_v)
                pltpu.async_copy(table_hbm.at[idx_v], rows_v, sem).wait()
                pltpu.sync_copy(rows_v, out_hbm.at[pl.ds(b, GCH)])

            return 0

        lax.fori_loop(0, nch, chunk, 0)

    def body_all(table_hbm, idx_hbm, out_hbm, idx_v, rows_v, sem):
        wid = lax.axis_index("s") * NC + lax.axis_index("c")
        base = wid * rpw

        def chunk(i, _):
            b = base + i * GCH
            pltpu.sync_copy(idx_hbm.at[pl.ds(b, GCH)], idx_v)
            pltpu.async_copy(table_hbm.at[idx_v], rows_v, sem).wait()
            pltpu.sync_copy(rows_v, out_hbm.at[pl.ds(b, GCH)])
            return 0

        lax.fori_loop(0, nch, chunk, 0)

    scratch = [
        pltpu.VMEM((GCH,), jnp.int32),
        pltpu.VMEM((GCH, D), jnp.float32),
    ]
    if meta is not None:
        return pl.kernel(
            body,
            out_type=jax.ShapeDtypeStruct((n_rows, D), jnp.float32),
            mesh=_SC_MESH(),
            scratch_types=scratch + [pltpu.VMEM((2 * L,), jnp.int32),
                                     pltpu.SemaphoreType.DMA],
            compiler_params=pltpu.CompilerParams(needs_layout_passes=False),
        )(table, idx, meta)
    return pl.kernel(
        body_all,
        out_type=jax.ShapeDtypeStruct((n_rows, D), jnp.float32),
        mesh=_SC_MESH(),
        scratch_types=scratch + [pltpu.SemaphoreType.DMA],
    )(table, idx)


def _mm_body(be_ref, ru_ref, x_ref, wg_ref, wu_ref, wd_ref, o_ref):
    i = pl.program_id(0)
    j = pl.program_id(1)
    ru = ru_ref[i]

    @pl.when(ru > HB)
    def _():
        x = x_ref[...]
        g = jnp.dot(x, wg_ref[0], preferred_element_type=jnp.float32)
        u = jnp.dot(x, wu_ref[0], preferred_element_type=jnp.float32)
        act = g * jax.nn.sigmoid(g) * u
        p = jnp.dot(act, wd_ref[0], preferred_element_type=jnp.float32)

        @pl.when(j == 0)
        def _():
            o_ref[...] = p

        @pl.when(j > 0)
        def _():
            o_ref[...] += p

    @pl.when((ru > 0) & (ru <= HB))
    def _():
        x = x_ref[0:HB, :]
        g = jnp.dot(x, wg_ref[0], preferred_element_type=jnp.float32)
        u = jnp.dot(x, wu_ref[0], preferred_element_type=jnp.float32)
        act = g * jax.nn.sigmoid(g) * u
        p = jnp.dot(act, wd_ref[0], preferred_element_type=jnp.float32)

        @pl.when(j == 0)
        def _():
            o_ref[0:HB, :] = p

        @pl.when(j > 0)
        def _():
            o_ref[0:HB, :] += p


def _grouped_mlp(xpad, gate_up, down, block_expert, rows_used):
    def live(ru, i, v, dead):
        return jnp.where(ru[i] > 0, v, dead)

    grid_spec = pltpu.PrefetchScalarGridSpec(
        num_scalar_prefetch=2,
        grid=(NBMAX, NJ),
        in_specs=[
            pl.BlockSpec((BLK, D),
                         lambda i, j, be, ru: (live(ru, i, i, 0), 0)),
            pl.BlockSpec((1, D, IBLK),
                         lambda i, j, be, ru: (be[i], 0, live(ru, i, j, 0))),
            pl.BlockSpec((1, D, IBLK),
                         lambda i, j, be, ru: (be[i], 0, NJ + live(ru, i, j, 0))),
            pl.BlockSpec((1, IBLK, D),
                         lambda i, j, be, ru: (be[i], live(ru, i, j, 0), 0)),
        ],
        out_specs=pl.BlockSpec((BLK, D), lambda i, j, be, ru: (i, 0)),
    )
    return pl.pallas_call(
        _mm_body,
        grid_spec=grid_spec,
        out_shape=jax.ShapeDtypeStruct((TPAD, D), jnp.float32),
        compiler_params=pltpu.CompilerParams(
            dimension_semantics=("arbitrary", "arbitrary"),
        ),
    )(block_expert, rows_used, xpad, gate_up, gate_up, down)


@jax.jit
def kernel(hidden, expert_ids, gate_up, down):
    ids = expert_ids.astype(jnp.int32)
    pos, src, meta = _route(ids)
    xpad = _gather_rows(hidden, src, TPAD, meta=meta)
    opad = _grouped_mlp(xpad, gate_up, down, meta[:NBMAX], meta[L:L + NBMAX])
    return _gather_rows(opad, pos, T)


# speedup vs baseline: 3.1304x; 1.0706x over previous
import functools

import jax
import jax.numpy as jnp
from jax import lax
from jax.experimental import pallas as pl
from jax.experimental.pallas import tpu as pltpu
from jax.experimental.pallas import tpu_sc as plsc

E = 8
D = 2048
I = 2048
T = 4096

BLK = 1024
BLK_SHIFT = 10
HB = 512
IBLK = 256
NJ = I // IBLK
NBMAX = T // BLK + E
TPAD = NBMAX * BLK

NC = 2
NS = 16
NW = NC * NS
L = 16
GCH = 32

_SC_MESH = functools.partial(
    plsc.VectorSubcoreMesh, core_axis_name="c", subcore_axis_name="s"
)


def _route_body(ids_hbm, pos_hbm, src_hbm, be_hbm,
                ids_v, rank_v, pos_v, src_v, run_v, off_v, be_v):
    wid = lax.axis_index("s") * NC + lax.axis_index("c")

    @pl.when(wid == 0)
    def _():
        pltpu.sync_copy(ids_hbm, ids_v)
        lane = lax.iota(jnp.int32, L)
        zeros = jnp.zeros((L,), jnp.int32)
        run_v[...] = zeros

        def p1(c, _):
            v = ids_v[pl.ds(c * L, L)]
            base = plsc.load_gather(run_v, [v])
            dup, last = plsc.scan_count(v)
            rank_v[pl.ds(c * L, L)] = base + dup - 1
            plsc.store_scatter(run_v, [v], base + dup, mask=last)
            return 0

        lax.fori_loop(0, T // L, p1, 0)

        counts = run_v[...]
        padded = ((counts + (BLK - 1)) >> BLK_SHIFT) << BLK_SHIFT
        end = plsc.cumsum(padded)
        off_v[...] = end - padded

        def z(c, _):
            src_v[pl.ds(c * L, L)] = (lane + c * L) & (T - 1)
            return 0

        lax.fori_loop(0, TPAD // L, z, 0)

        def p2(c, _):
            v = ids_v[pl.ds(c * L, L)]
            offs = plsc.load_gather(off_v, [v])
            p = offs + rank_v[pl.ds(c * L, L)]
            pos_v[pl.ds(c * L, L)] = p
            plsc.store_scatter(src_v, [p], lane + c * L)
            return 0

        lax.fori_loop(0, T // L, p2, 0)

        nb = jnp.sum(end * jnp.where(lane == E - 1, 1, 0)) >> BLK_SHIFT
        off_v[...] = (end - padded) + counts
        blkpos = lane * BLK
        bev = zeros
        for e in range(E):
            end_e = jnp.sum(jnp.where(lane == e, end, 0))
            bev = bev + jnp.where(blkpos >= end_e, 1, 0)
        bev = jnp.minimum(bev, E - 1)
        realend = plsc.load_gather(off_v, [bev])
        used = jnp.minimum(jnp.maximum(realend - blkpos, 0), BLK)
        be_v[pl.ds(0, L)] = jnp.where(lane == NBMAX, nb, bev)
        be_v[pl.ds(L, L)] = used
        pltpu.sync_copy(pos_v, pos_hbm)
        pltpu.sync_copy(src_v, src_hbm)
        pltpu.sync_copy(be_v, be_hbm)


def _route(ids):
    return pl.kernel(
        _route_body,
        out_type=(
            jax.ShapeDtypeStruct((T,), jnp.int32),
            jax.ShapeDtypeStruct((TPAD,), jnp.int32),
            jax.ShapeDtypeStruct((2 * L,), jnp.int32),
        ),
        mesh=_SC_MESH(),
        scratch_types=[
            pltpu.VMEM((T,), jnp.int32),
            pltpu.VMEM((T,), jnp.int32),
            pltpu.VMEM((T,), jnp.int32),
            pltpu.VMEM((TPAD,), jnp.int32),
            pltpu.VMEM((L,), jnp.int32),
            pltpu.VMEM((L,), jnp.int32),
            pltpu.VMEM((2 * L,), jnp.int32),
        ],
        compiler_params=pltpu.CompilerParams(needs_layout_passes=False),
    )(ids)


def _gather_rows(table, idx, n_rows, meta=None):
    rpw = n_rows // NW
    nch = rpw // GCH

    def body(table_hbm, idx_hbm, meta_hbm, out_hbm, idx_v, rows_v, m_v, sem):
        wid = lax.axis_index("s") * NC + lax.axis_index("c")
        lane = lax.iota(jnp.int32, L)
        pltpu.sync_copy(meta_hbm, m_v)
        nbrows = jnp.sum(jnp.where(lane == NBMAX, m_v[pl.ds(0, L)], 0)) * BLK

        def chunk(i, _):
            b = (i * NW + wid) * GCH

            @pl.when(b < nbrows)
            def _():
                pltpu.sync_copy(idx_hbm.at[pl.ds(b, GCH)], idx_v)
                pltpu.async_copy(table_hbm.at[idx_v], rows_v, sem).wait()
                pltpu.sync_copy(rows_v, out_hbm.at[pl.ds(b, GCH)])

            return 0

        lax.fori_loop(0, nch, chunk, 0)

    def body_all(table_hbm, idx_hbm, out_hbm, idx_v, rows_v, sem):
        wid = lax.axis_index("s") * NC + lax.axis_index("c")
        base = wid * rpw

        def chunk(i, _):
            b = base + i * GCH
            pltpu.sync_copy(idx_hbm.at[pl.ds(b, GCH)], idx_v)
            pltpu.async_copy(table_hbm.at[idx_v], rows_v, sem).wait()
            pltpu.sync_copy(rows_v, out_hbm.at[pl.ds(b, GCH)])
            return 0

        lax.fori_loop(0, nch, chunk, 0)

    scratch = [
        pltpu.VMEM((GCH,), jnp.int32),
        pltpu.VMEM((GCH, D), jnp.float32),
    ]
    if meta is not None:
        return pl.kernel(
            body,
            out_type=jax.ShapeDtypeStruct((n_rows, D), jnp.float32),
            mesh=_SC_MESH(),
            scratch_types=scratch + [pltpu.VMEM((2 * L,), jnp.int32),
                                     pltpu.SemaphoreType.DMA],
            compiler_params=pltpu.CompilerParams(needs_layout_passes=False),
        )(table, idx, meta)
    return pl.kernel(
        body_all,
        out_type=jax.ShapeDtypeStruct((n_rows, D), jnp.float32),
        mesh=_SC_MESH(),
        scratch_types=scratch + [pltpu.SemaphoreType.DMA],
    )(table, idx)


def _mm_body(be_ref, ru_ref, x_ref, wg_ref, wu_ref, wd_ref, o_ref):
    i = pl.program_id(0)
    j = pl.program_id(1)
    ru = ru_ref[i]

    @pl.when(ru > HB)
    def _():
        x = x_ref[...]
        g = jnp.dot(x, wg_ref[0], preferred_element_type=jnp.float32)
        u = jnp.dot(x, wu_ref[0], preferred_element_type=jnp.float32)
        act = g * jax.nn.sigmoid(g) * u
        p = jnp.dot(act, wd_ref[0], preferred_element_type=jnp.float32)

        @pl.when(j == 0)
        def _():
            o_ref[...] = p

        @pl.when(j > 0)
        def _():
            o_ref[...] += p

    @pl.when((ru > 0) & (ru <= HB))
    def _():
        x = x_ref[0:HB, :]
        g = jnp.dot(x, wg_ref[0], preferred_element_type=jnp.float32)
        u = jnp.dot(x, wu_ref[0], preferred_element_type=jnp.float32)
        act = g * jax.nn.sigmoid(g) * u
        p = jnp.dot(act, wd_ref[0], preferred_element_type=jnp.float32)

        @pl.when(j == 0)
        def _():
            o_ref[0:HB, :] = p

        @pl.when(j > 0)
        def _():
            o_ref[0:HB, :] += p


def _grouped_mlp(xpad, gate_up, down, block_expert, rows_used):
    def live(ru, i, v, dead):
        return jnp.where(ru[i] > 0, v, dead)

    grid_spec = pltpu.PrefetchScalarGridSpec(
        num_scalar_prefetch=2,
        grid=(NBMAX, NJ),
        in_specs=[
            pl.BlockSpec((BLK, D),
                         lambda i, j, be, ru: (live(ru, i, i, 0), 0)),
            pl.BlockSpec((1, D, IBLK),
                         lambda i, j, be, ru: (be[i], 0, live(ru, i, j, 0))),
            pl.BlockSpec((1, D, IBLK),
                         lambda i, j, be, ru: (be[i], 0, NJ + live(ru, i, j, 0))),
            pl.BlockSpec((1, IBLK, D),
                         lambda i, j, be, ru: (be[i], live(ru, i, j, 0), 0)),
        ],
        out_specs=pl.BlockSpec(
            (BLK, D), lambda i, j, be, ru: (live(ru, i, i, NBMAX), 0)
        ),
    )
    return pl.pallas_call(
        _mm_body,
        grid_spec=grid_spec,
        out_shape=jax.ShapeDtypeStruct((TPAD + BLK, D), jnp.float32),
        compiler_params=pltpu.CompilerParams(
            dimension_semantics=("arbitrary", "arbitrary"),
        ),
    )(block_expert, rows_used, xpad, gate_up, gate_up, down)


@jax.jit
def kernel(hidden, expert_ids, gate_up, down):
    ids = expert_ids.astype(jnp.int32)
    pos, src, meta = _route(ids)
    xpad = _gather_rows(hidden, src, TPAD, meta=meta)
    opad = _grouped_mlp(xpad, gate_up, down, meta[:NBMAX], meta[L:L + NBMAX])
    return _gather_rows(opad, pos, T)


# speedup vs baseline: 3.1769x; 1.0149x over previous
import functools

import jax
import jax.numpy as jnp
from jax import lax
from jax.experimental import pallas as pl
from jax.experimental.pallas import tpu as pltpu
from jax.experimental.pallas import tpu_sc as plsc

E = 8
D = 2048
I = 2048
T = 4096

BLK = 1024
BLK_SHIFT = 10
HB = 512
IBLK = 256
NJ = I // IBLK
NBMAX = T // BLK + E
TPAD = NBMAX * BLK

NC = 2
NS = 16
NW = NC * NS
L = 16
GCH = 32

_SC_MESH = functools.partial(
    plsc.VectorSubcoreMesh, core_axis_name="c", subcore_axis_name="s"
)


def _route_body(ids_hbm, pos_hbm, src_hbm, be_hbm,
                ids_v, rank_v, pos_v, src_v, run_v, off_v, be_v):
    wid = lax.axis_index("s") * NC + lax.axis_index("c")

    @pl.when(wid == 0)
    def _():
        pltpu.sync_copy(ids_hbm, ids_v)
        lane = lax.iota(jnp.int32, L)
        zeros = jnp.zeros((L,), jnp.int32)
        run_v[...] = zeros

        def p1(c, _):
            v = ids_v[pl.ds(c * L, L)]
            base = plsc.load_gather(run_v, [v])
            dup, last = plsc.scan_count(v)
            rank_v[pl.ds(c * L, L)] = base + dup - 1
            plsc.store_scatter(run_v, [v], base + dup, mask=last)
            return 0

        lax.fori_loop(0, T // L, p1, 0)

        counts = run_v[...]
        padded = ((counts + (BLK - 1)) >> BLK_SHIFT) << BLK_SHIFT
        end = plsc.cumsum(padded)
        off_v[...] = end - padded

        def z(c, _):
            src_v[pl.ds(c * L, L)] = (lane + c * L) & (T - 1)
            return 0

        lax.fori_loop(0, TPAD // L, z, 0)

        def p2(c, _):
            v = ids_v[pl.ds(c * L, L)]
            offs = plsc.load_gather(off_v, [v])
            p = offs + rank_v[pl.ds(c * L, L)]
            pos_v[pl.ds(c * L, L)] = p
            plsc.store_scatter(src_v, [p], lane + c * L)
            return 0

        lax.fori_loop(0, T // L, p2, 0)

        nb = jnp.sum(end * jnp.where(lane == E - 1, 1, 0)) >> BLK_SHIFT
        off_v[...] = (end - padded) + counts
        blkpos = lane * BLK
        bev = zeros
        for e in range(E):
            end_e = jnp.sum(jnp.where(lane == e, end, 0))
            bev = bev + jnp.where(blkpos >= end_e, 1, 0)
        bev = jnp.minimum(bev, E - 1)
        realend = plsc.load_gather(off_v, [bev])
        used = jnp.minimum(jnp.maximum(realend - blkpos, 0), BLK)
        be_v[pl.ds(0, L)] = jnp.where(lane == NBMAX, nb, bev)
        be_v[pl.ds(L, L)] = used
        pltpu.sync_copy(pos_v, pos_hbm)
        pltpu.sync_copy(src_v, src_hbm)
        pltpu.sync_copy(be_v, be_hbm)


def _route(ids):
    return pl.kernel(
        _route_body,
        out_type=(
            jax.ShapeDtypeStruct((T,), jnp.int32),
            jax.ShapeDtypeStruct((TPAD,), jnp.int32),
            jax.ShapeDtypeStruct((2 * L,), jnp.int32),
        ),
        mesh=_SC_MESH(),
        scratch_types=[
            pltpu.VMEM((T,), jnp.int32),
            pltpu.VMEM((T,), jnp.int32),
            pltpu.VMEM((T,), jnp.int32),
            pltpu.VMEM((TPAD,), jnp.int32),
            pltpu.VMEM((L,), jnp.int32),
            pltpu.VMEM((L,), jnp.int32),
            pltpu.VMEM((2 * L,), jnp.int32),
        ],
        compiler_params=pltpu.CompilerParams(needs_layout_passes=False),
    )(ids)


def _gather_rows(table, idx, n_rows, meta=None):
    rpw = n_rows // NW
    nch = rpw // GCH

    def body(table_hbm, idx_hbm, meta_hbm, out_hbm, idx_v, rows_v, m_v, sem):
        wid = lax.axis_index("s") * NC + lax.axis_index("c")
        lane = lax.iota(jnp.int32, L)
        pltpu.sync_copy(meta_hbm, m_v)
        used_vec = m_v[pl.ds(L, L)]

        def chunk(i, _):
            b = (i * NW + wid) * GCH
            ival = b >> BLK_SHIFT
            thr = jnp.sum(jnp.where(lane == ival, used_vec, 0))

            @pl.when(b - ival * BLK < thr)
            def _():
                pltpu.sync_copy(idx_hbm.at[pl.ds(b, GCH)], idx_v)
                pltpu.async_copy(table_hbm.at[idx_v], rows_v, sem).wait()
                pltpu.sync_copy(rows_v, out_hbm.at[pl.ds(b, GCH)])

            return 0

        lax.fori_loop(0, nch, chunk, 0)

    def body_all(table_hbm, idx_hbm, out_hbm, idx_v, rows_v, sem):
        wid = lax.axis_index("s") * NC + lax.axis_index("c")
        base = wid * rpw

        def chunk(i, _):
            b = base + i * GCH
            pltpu.sync_copy(idx_hbm.at[pl.ds(b, GCH)], idx_v)
            pltpu.async_copy(table_hbm.at[idx_v], rows_v, sem).wait()
            pltpu.sync_copy(rows_v, out_hbm.at[pl.ds(b, GCH)])
            return 0

        lax.fori_loop(0, nch, chunk, 0)

    scratch = [
        pltpu.VMEM((GCH,), jnp.int32),
        pltpu.VMEM((GCH, D), jnp.float32),
    ]
    if meta is not None:
        return pl.kernel(
            body,
            out_type=jax.ShapeDtypeStruct((n_rows, D), jnp.float32),
            mesh=_SC_MESH(),
            scratch_types=scratch + [pltpu.VMEM((2 * L,), jnp.int32),
                                     pltpu.SemaphoreType.DMA],
            compiler_params=pltpu.CompilerParams(needs_layout_passes=False),
        )(table, idx, meta)
    return pl.kernel(
        body_all,
        out_type=jax.ShapeDtypeStruct((n_rows, D), jnp.float32),
        mesh=_SC_MESH(),
        scratch_types=scratch + [pltpu.SemaphoreType.DMA],
    )(table, idx)


def _mm_body(be_ref, ru_ref, x_ref, wg_ref, wu_ref, wd_ref, o_ref):
    i = pl.program_id(0)
    j = pl.program_id(1)
    ru = ru_ref[i]

    @pl.when(ru > HB)
    def _():
        x = x_ref[...]
        g = jnp.dot(x, wg_ref[0], preferred_element_type=jnp.float32)
        u = jnp.dot(x, wu_ref[0], preferred_element_type=jnp.float32)
        act = g * jax.nn.sigmoid(g) * u
        p = jnp.dot(act, wd_ref[0], preferred_element_type=jnp.float32)

        @pl.when(j == 0)
        def _():
            o_ref[...] = p

        @pl.when(j > 0)
        def _():
            o_ref[...] += p

    @pl.when((ru > 0) & (ru <= HB))
    def _():
        x = x_ref[0:HB, :]
        g = jnp.dot(x, wg_ref[0], preferred_element_type=jnp.float32)
        u = jnp.dot(x, wu_ref[0], preferred_element_type=jnp.float32)
        act = g * jax.nn.sigmoid(g) * u
        p = jnp.dot(act, wd_ref[0], preferred_element_type=jnp.float32)

        @pl.when(j == 0)
        def _():
            o_ref[0:HB, :] = p

        @pl.when(j > 0)
        def _():
            o_ref[0:HB, :] += p


def _grouped_mlp(xpad, gate_up, down, block_expert, rows_used):
    def live(ru, i, v, dead):
        return jnp.where(ru[i] > 0, v, dead)

    grid_spec = pltpu.PrefetchScalarGridSpec(
        num_scalar_prefetch=2,
        grid=(NBMAX, NJ),
        in_specs=[
            pl.BlockSpec((BLK, D),
                         lambda i, j, be, ru: (live(ru, i, i, 0), 0)),
            pl.BlockSpec((1, D, IBLK),
                         lambda i, j, be, ru: (be[i], 0, live(ru, i, j, 0))),
            pl.BlockSpec((1, D, IBLK),
                         lambda i, j, be, ru: (be[i], 0, NJ + live(ru, i, j, 0))),
            pl.BlockSpec((1, IBLK, D),
                         lambda i, j, be, ru: (be[i], live(ru, i, j, 0), 0)),
        ],
        out_specs=pl.BlockSpec(
            (BLK, D), lambda i, j, be, ru: (live(ru, i, i, NBMAX), 0)
        ),
    )
    return pl.pallas_call(
        _mm_body,
        grid_spec=grid_spec,
        out_shape=jax.ShapeDtypeStruct((TPAD + BLK, D), jnp.float32),
        compiler_params=pltpu.CompilerParams(
            dimension_semantics=("arbitrary", "arbitrary"),
        ),
    )(block_expert, rows_used, xpad, gate_up, gate_up, down)


@jax.jit
def kernel(hidden, expert_ids, gate_up, down):
    ids = expert_ids.astype(jnp.int32)
    pos, src, meta = _route(ids)
    xpad = _gather_rows(hidden, src, TPAD, meta=meta)
    opad = _grouped_mlp(xpad, gate_up, down, meta[:NBMAX], meta[L:L + NBMAX])
    return _gather_rows(opad, pos, T)
